# baseline (speedup 1.0000x reference)
import jax
import jax.numpy as jnp
from jax import lax
from jax.experimental import pallas as pl
from jax.experimental.pallas import tpu as pltpu
from jax.experimental.pallas import tpu_sc as plsc

D = 128
HALF = 64
HW = 900
N_PIX = 256 * HW
NUM_COLORS = 10
NW = 32
PPW = N_PIX // NW
CHUNK = 160
N_CHUNKS = PPW // CHUNK
NBUF = 3
GROUPS = CHUNK // 16


def _sc_kernel(x_hbm, idx_hbm, sp_hbm, ch_hbm, out_hbm,
               x_v0, x_v1, x_v2, i_v0, i_v1, i_v2, sp_v, ch_v,
               ld0, ld1, ld2, st0, st1, st2):
    x_bufs = (x_v0, x_v1, x_v2)
    i_bufs = (i_v0, i_v1, i_v2)
    ld_sems = (ld0, ld1, ld2)
    st_sems = (st0, st1, st2)

    wid = lax.axis_index("s") * 2 + lax.axis_index("c")
    base = wid * PPW

    pltpu.sync_copy(sp_hbm, sp_v)
    pltpu.sync_copy(ch_hbm, ch_v)

    lane = lax.iota(jnp.int32, 16)

    def start_load(c, b):
        pstart = base + c * CHUNK
        pltpu.async_copy(x_hbm.at[pl.ds(pstart, CHUNK)],
                         x_bufs[b], ld_sems[b])
        pltpu.async_copy(idx_hbm.at[pl.ds(pstart, CHUNK)],
                         i_bufs[b], ld_sems[b])

    def wait_load(c, b):
        pstart = base + c * CHUNK
        pltpu.make_async_copy(x_hbm.at[pl.ds(pstart, CHUNK)],
                              x_bufs[b], ld_sems[b]).wait()
        pltpu.make_async_copy(idx_hbm.at[pl.ds(pstart, CHUNK)],
                              i_bufs[b], ld_sems[b]).wait()

    def start_store(c, b):
        pstart = base + c * CHUNK
        pltpu.async_copy(x_bufs[b],
                         out_hbm.at[pl.ds(pstart, CHUNK)],
                         st_sems[b])

    def wait_store(c, b):
        pstart = base + c * CHUNK
        pltpu.make_async_copy(x_bufs[b],
                              out_hbm.at[pl.ds(pstart, CHUNK)],
                              st_sems[b]).wait()

    for b in range(NBUF):
        start_load(b, b)

    def chunk_compute(pstart, x_v, idx_v):
        def group_body(g, _):
            p16 = g * 16
            cidx = idx_v[pl.ds(p16, 16)]
            for p in range(16):
                ploc = p16 + p
                prow = lax.rem(pstart + ploc, HW)
                sprow = prow // 2
                rcol = lax.rem(prow, 2) * HALF
                for j in range(HALF // 16):
                    o = j * 16
                    x_v[ploc, pl.ds(o, 16)] = (
                        x_v[ploc, pl.ds(o, 16)]
                        + sp_v[sprow, pl.ds(rcol + o, 16)])
                cbase = cidx[jnp.full((16,), p, jnp.int32)] * HALF
                for j in range(HALF // 16):
                    o = j * 16
                    cv = plsc.load_gather(ch_v, [cbase + o + lane])
                    x_v[ploc, pl.ds(HALF + o, 16)] = (
                        x_v[ploc, pl.ds(HALF + o, 16)] + cv)
            return 0

        lax.fori_loop(0, GROUPS, group_body, 0)

    def iter_body(k, _):
        for b in range(NBUF):
            c = k * NBUF + b
            wait_load(c, b)
            chunk_compute(base + c * CHUNK, x_bufs[b], i_bufs[b])
            start_store(c, b)
            @pl.when(c + NBUF < N_CHUNKS)
            def _():
                wait_store(c, b)
                start_load(c + NBUF, b)
        return 0

    lax.fori_loop(0, N_CHUNKS // NBUF, iter_body, 0)

    for b in range(NBUF):
        wait_store(N_CHUNKS - NBUF + b, b)


def kernel(x, color_indices, spatial_pe, chromatic_pe):
    Bb, Hh, Ww, d = x.shape
    xf = x.reshape(N_PIX, D)
    idxf = color_indices.astype(jnp.int32).reshape(N_PIX)
    spf = spatial_pe[:Hh, :Ww, :].reshape(HW // 2, D)
    chf = chromatic_pe.reshape(NUM_COLORS * HALF)

    mesh = plsc.VectorSubcoreMesh(core_axis_name="c", subcore_axis_name="s")
    run = pl.kernel(
        _sc_kernel,
        jax.ShapeDtypeStruct((N_PIX, D), jnp.float32),
        mesh=mesh,
        compiler_params=pltpu.CompilerParams(needs_layout_passes=False),
        scratch_types=[
            pltpu.VMEM((CHUNK, D), jnp.float32),
            pltpu.VMEM((CHUNK, D), jnp.float32),
            pltpu.VMEM((CHUNK, D), jnp.float32),
            pltpu.VMEM((CHUNK,), jnp.int32),
            pltpu.VMEM((CHUNK,), jnp.int32),
            pltpu.VMEM((CHUNK,), jnp.int32),
            pltpu.VMEM((HW // 2, D), jnp.float32),
            pltpu.VMEM((NUM_COLORS * HALF,), jnp.float32),
            pltpu.SemaphoreType.DMA,
            pltpu.SemaphoreType.DMA,
            pltpu.SemaphoreType.DMA,
            pltpu.SemaphoreType.DMA,
            pltpu.SemaphoreType.DMA,
            pltpu.SemaphoreType.DMA,
        ],
    )
    out = run(xf, idxf, spf, chf)
    return out.reshape(Bb, Hh, Ww, d)


# speedup vs baseline: 1.0006x; 1.0006x over previous
import jax
import jax.numpy as jnp
from jax import lax
from jax.experimental import pallas as pl
from jax.experimental.pallas import tpu as pltpu
from jax.experimental.pallas import tpu_sc as plsc

D = 128
HALF = 64
HW = 900
N_PIX = 256 * HW
NUM_COLORS = 10
NW = 32
PPW = N_PIX // NW
CHUNK = 160
N_CHUNKS = PPW // CHUNK
NBUF = 3
GROUPS = CHUNK // 16


def _sc_kernel(x_hbm, idx_hbm, sp_hbm, ch_hbm, out_hbm,
               x_v0, x_v1, x_v2, i_v0, i_v1, i_v2, sp_v, ch_v,
               ld0, ld1, ld2, st0, st1, st2):
    x_bufs = (x_v0, x_v1, x_v2)
    i_bufs = (i_v0, i_v1, i_v2)
    ld_sems = (ld0, ld1, ld2)
    st_sems = (st0, st1, st2)

    wid = lax.axis_index("s") * 2 + lax.axis_index("c")
    base = wid * PPW

    pltpu.sync_copy(sp_hbm, sp_v)
    pltpu.sync_copy(ch_hbm, ch_v)

    lane = lax.iota(jnp.int32, 16)

    HC = CHUNK // 2

    def start_load(c, b):
        pstart = base + c * CHUNK
        pltpu.async_copy(x_hbm.at[pl.ds(pstart, HC)],
                         x_bufs[b].at[pl.ds(0, HC)], ld_sems[b])
        pltpu.async_copy(x_hbm.at[pl.ds(pstart + HC, HC)],
                         x_bufs[b].at[pl.ds(HC, HC)], st_sems[b])
        pltpu.async_copy(idx_hbm.at[pl.ds(pstart, CHUNK)],
                         i_bufs[b], ld_sems[b])

    def wait_load(c, b):
        pstart = base + c * CHUNK
        pltpu.make_async_copy(x_hbm.at[pl.ds(pstart, HC)],
                              x_bufs[b].at[pl.ds(0, HC)], ld_sems[b]).wait()
        pltpu.make_async_copy(x_hbm.at[pl.ds(pstart + HC, HC)],
                              x_bufs[b].at[pl.ds(HC, HC)], st_sems[b]).wait()
        pltpu.make_async_copy(idx_hbm.at[pl.ds(pstart, CHUNK)],
                              i_bufs[b], ld_sems[b]).wait()

    def start_store(c, b):
        pstart = base + c * CHUNK
        pltpu.async_copy(x_bufs[b].at[pl.ds(0, HC)],
                         out_hbm.at[pl.ds(pstart, HC)], st_sems[b])
        pltpu.async_copy(x_bufs[b].at[pl.ds(HC, HC)],
                         out_hbm.at[pl.ds(pstart + HC, HC)], ld_sems[b])

    def wait_store(c, b):
        pstart = base + c * CHUNK
        pltpu.make_async_copy(x_bufs[b].at[pl.ds(0, HC)],
                              out_hbm.at[pl.ds(pstart, HC)],
                              st_sems[b]).wait()
        pltpu.make_async_copy(x_bufs[b].at[pl.ds(HC, HC)],
                              out_hbm.at[pl.ds(pstart + HC, HC)],
                              ld_sems[b]).wait()

    for b in range(NBUF):
        start_load(b, b)

    def chunk_compute(pstart, x_v, idx_v):
        def group_body(g, _):
            p16 = g * 16
            cidx = idx_v[pl.ds(p16, 16)]
            for p in range(16):
                ploc = p16 + p
                prow = lax.rem(pstart + ploc, HW)
                sprow = prow // 2
                rcol = lax.rem(prow, 2) * HALF
                for j in range(HALF // 16):
                    o = j * 16
                    x_v[ploc, pl.ds(o, 16)] = (
                        x_v[ploc, pl.ds(o, 16)]
                        + sp_v[sprow, pl.ds(rcol + o, 16)])
                cbase = cidx[jnp.full((16,), p, jnp.int32)] * HALF
                for j in range(HALF // 16):
                    o = j * 16
                    cv = plsc.load_gather(ch_v, [cbase + o + lane])
                    x_v[ploc, pl.ds(HALF + o, 16)] = (
                        x_v[ploc, pl.ds(HALF + o, 16)] + cv)
            return 0

        lax.fori_loop(0, GROUPS, group_body, 0)

    def iter_body(k, _):
        for b in range(NBUF):
            c = k * NBUF + b
            wait_load(c, b)
            chunk_compute(base + c * CHUNK, x_bufs[b], i_bufs[b])
            start_store(c, b)
            @pl.when(c + NBUF < N_CHUNKS)
            def _():
                wait_store(c, b)
                start_load(c + NBUF, b)
        return 0

    lax.fori_loop(0, N_CHUNKS // NBUF, iter_body, 0)

    for b in range(NBUF):
        wait_store(N_CHUNKS - NBUF + b, b)


def kernel(x, color_indices, spatial_pe, chromatic_pe):
    Bb, Hh, Ww, d = x.shape
    xf = x.reshape(N_PIX, D)
    idxf = color_indices.astype(jnp.int32).reshape(N_PIX)
    spf = spatial_pe[:Hh, :Ww, :].reshape(HW // 2, D)
    chf = chromatic_pe.reshape(NUM_COLORS * HALF)

    mesh = plsc.VectorSubcoreMesh(core_axis_name="c", subcore_axis_name="s")
    run = pl.kernel(
        _sc_kernel,
        jax.ShapeDtypeStruct((N_PIX, D), jnp.float32),
        mesh=mesh,
        compiler_params=pltpu.CompilerParams(needs_layout_passes=False),
        scratch_types=[
            pltpu.VMEM((CHUNK, D), jnp.float32),
            pltpu.VMEM((CHUNK, D), jnp.float32),
            pltpu.VMEM((CHUNK, D), jnp.float32),
            pltpu.VMEM((CHUNK,), jnp.int32),
            pltpu.VMEM((CHUNK,), jnp.int32),
            pltpu.VMEM((CHUNK,), jnp.int32),
            pltpu.VMEM((HW // 2, D), jnp.float32),
            pltpu.VMEM((NUM_COLORS * HALF,), jnp.float32),
            pltpu.SemaphoreType.DMA,
            pltpu.SemaphoreType.DMA,
            pltpu.SemaphoreType.DMA,
            pltpu.SemaphoreType.DMA,
            pltpu.SemaphoreType.DMA,
            pltpu.SemaphoreType.DMA,
        ],
    )
    out = run(xf, idxf, spf, chf)
    return out.reshape(Bb, Hh, Ww, d)


# speedup vs baseline: 1.1363x; 1.1356x over previous
import jax
import jax.numpy as jnp
from jax import lax
from jax.experimental import pallas as pl
from jax.experimental.pallas import tpu as pltpu
from jax.experimental.pallas import tpu_sc as plsc

D = 128
HALF = 64
HW = 900
NUM_COLORS = 10
COLORS_PAD = 16
NW = 32

TC_IMGS = 128
SC_IMGS = 128
SC_PIX = SC_IMGS * HW
PPW = SC_PIX // NW
CHUNK = 80
N_CHUNKS = PPW // CHUNK
NBUF = 3
GROUPS = CHUNK // 16

B_BLK = 8


def _tc_body(x_ref, idx_ref, sp_ref, ch_ref, out_ref):
    x = x_ref[...]
    idx = idx_ref[...]
    sp = sp_ref[...]
    ch = ch_ref[...]
    b_blk, hw = idx.shape
    lanes = jax.lax.broadcasted_iota(jnp.int32, (b_blk, hw, COLORS_PAD), 2)
    onehot = (idx[..., None] == lanes).astype(jnp.float32)
    base = x + sp[None, :, :]
    for b in range(b_blk):
        out_ref[b] = base[b] + jnp.dot(
            onehot[b], ch, preferred_element_type=jnp.float32)


def _tc_half(xf, idxf, sp128, ch128, n_imgs):
    grid = (n_imgs // B_BLK,)
    return pl.pallas_call(
        _tc_body,
        grid=grid,
        in_specs=[
            pl.BlockSpec((B_BLK, HW, D), lambda i: (i, 0, 0)),
            pl.BlockSpec((B_BLK, HW), lambda i: (i, 0)),
            pl.BlockSpec((HW, D), lambda i: (0, 0)),
            pl.BlockSpec((COLORS_PAD, D), lambda i: (0, 0)),
        ],
        out_specs=pl.BlockSpec((B_BLK, HW, D), lambda i: (i, 0, 0)),
        out_shape=jax.ShapeDtypeStruct((n_imgs, HW, D), jnp.float32),
    )(xf, idxf, sp128, ch128)


def _sc_kernel(x_hbm, idx_hbm, sp_hbm, ch_hbm, out_hbm,
               x_v0, x_v1, x_v2, i_v0, i_v1, i_v2, sp_v, ch_v,
               ld0, ld1, ld2, st0, st1, st2):
    x_bufs = (x_v0, x_v1, x_v2)
    i_bufs = (i_v0, i_v1, i_v2)
    ld_sems = (ld0, ld1, ld2)
    st_sems = (st0, st1, st2)

    wid = lax.axis_index("s") * 2 + lax.axis_index("c")
    base = wid * PPW

    pltpu.sync_copy(sp_hbm, sp_v)
    pltpu.sync_copy(ch_hbm, ch_v)

    lane = lax.iota(jnp.int32, 16)

    def start_load(c, b):
        pstart = base + c * CHUNK
        pltpu.async_copy(x_hbm.at[pl.ds(pstart, CHUNK)],
                         x_bufs[b], ld_sems[b])
        pltpu.async_copy(idx_hbm.at[pl.ds(pstart, CHUNK)],
                         i_bufs[b], ld_sems[b])

    def wait_load(c, b):
        pstart = base + c * CHUNK
        pltpu.make_async_copy(x_hbm.at[pl.ds(pstart, CHUNK)],
                              x_bufs[b], ld_sems[b]).wait()
        pltpu.make_async_copy(idx_hbm.at[pl.ds(pstart, CHUNK)],
                              i_bufs[b], ld_sems[b]).wait()

    def start_store(c, b):
        pstart = base + c * CHUNK
        pltpu.async_copy(x_bufs[b],
                         out_hbm.at[pl.ds(pstart, CHUNK)], st_sems[b])

    def wait_store(c, b):
        pstart = base + c * CHUNK
        pltpu.make_async_copy(x_bufs[b],
                              out_hbm.at[pl.ds(pstart, CHUNK)],
                              st_sems[b]).wait()

    for b in range(NBUF):
        start_load(b, b)

    def chunk_compute(pstart, x_v, idx_v):
        def group_body(g, _):
            p16 = g * 16
            cidx = idx_v[pl.ds(p16, 16)]
            for p in range(16):
                ploc = p16 + p
                prow = lax.rem(pstart + ploc, HW)
                sprow = prow // 2
                rcol = lax.rem(prow, 2) * HALF
                for j in range(HALF // 16):
                    o = j * 16
                    x_v[ploc, pl.ds(o, 16)] = (
                        x_v[ploc, pl.ds(o, 16)]
                        + sp_v[sprow, pl.ds(rcol + o, 16)])
                cbase = cidx[jnp.full((16,), p, jnp.int32)] * HALF
                for j in range(HALF // 16):
                    o = j * 16
                    cv = plsc.load_gather(ch_v, [cbase + o + lane])
                    x_v[ploc, pl.ds(HALF + o, 16)] = (
                        x_v[ploc, pl.ds(HALF + o, 16)] + cv)
            return 0

        lax.fori_loop(0, GROUPS, group_body, 0)

    def iter_body(k, _):
        for b in range(NBUF):
            c = k * NBUF + b
            wait_load(c, b)
            chunk_compute(base + c * CHUNK, x_bufs[b], i_bufs[b])
            start_store(c, b)

            @pl.when(c + NBUF < N_CHUNKS)
            def _():
                wait_store(c, b)
                start_load(c + NBUF, b)
        return 0

    lax.fori_loop(0, N_CHUNKS // NBUF, iter_body, 0)

    for b in range(NBUF):
        wait_store(N_CHUNKS - NBUF + b, b)


def _sc_half(xf, idxf, spf, chf):
    mesh = plsc.VectorSubcoreMesh(core_axis_name="c", subcore_axis_name="s")
    run = pl.kernel(
        _sc_kernel,
        jax.ShapeDtypeStruct((SC_PIX, D), jnp.float32),
        mesh=mesh,
        compiler_params=pltpu.CompilerParams(needs_layout_passes=False),
        scratch_types=[
            pltpu.VMEM((CHUNK, D), jnp.float32),
            pltpu.VMEM((CHUNK, D), jnp.float32),
            pltpu.VMEM((CHUNK, D), jnp.float32),
            pltpu.VMEM((CHUNK,), jnp.int32),
            pltpu.VMEM((CHUNK,), jnp.int32),
            pltpu.VMEM((CHUNK,), jnp.int32),
            pltpu.VMEM((HW // 2, D), jnp.float32),
            pltpu.VMEM((NUM_COLORS * HALF,), jnp.float32),
            pltpu.SemaphoreType.DMA,
            pltpu.SemaphoreType.DMA,
            pltpu.SemaphoreType.DMA,
            pltpu.SemaphoreType.DMA,
            pltpu.SemaphoreType.DMA,
            pltpu.SemaphoreType.DMA,
        ],
    )
    return run(xf, idxf, spf, chf)


def kernel(x, color_indices, spatial_pe, chromatic_pe):
    Bb, Hh, Ww, d = x.shape
    idx32 = color_indices.astype(jnp.int32)

    sp_half = spatial_pe[:Hh, :Ww, :].reshape(HW, HALF)

    sp128 = jnp.concatenate(
        [sp_half, jnp.zeros((HW, HALF), jnp.float32)], axis=-1)
    ch128 = jnp.zeros((COLORS_PAD, D), jnp.float32)
    ch128 = ch128.at[:NUM_COLORS, HALF:].set(chromatic_pe)
    out_tc = _tc_half(
        x[:TC_IMGS].reshape(TC_IMGS, HW, D),
        idx32[:TC_IMGS].reshape(TC_IMGS, HW),
        sp128, ch128, TC_IMGS)

    out_sc = _sc_half(
        x[TC_IMGS:].reshape(SC_PIX, D),
        idx32[TC_IMGS:].reshape(SC_PIX),
        sp_half.reshape(HW // 2, D),
        chromatic_pe.reshape(NUM_COLORS * HALF))

    out = jnp.concatenate(
        [out_tc.reshape(TC_IMGS, HW, D),
         out_sc.reshape(SC_IMGS, HW, D)], axis=0)
    return out.reshape(Bb, Hh, Ww, d)


# speedup vs baseline: 1.3574x; 1.1945x over previous
import jax
import jax.numpy as jnp
from jax import lax
from jax.experimental import pallas as pl
from jax.experimental.pallas import tpu as pltpu
from jax.experimental.pallas import tpu_sc as plsc

D = 128
HALF = 64
HW = 900
NUM_COLORS = 10
COLORS_PAD = 16
NW = 32

TC_IMGS = 192
SC_IMGS = 64
SC_PIX = SC_IMGS * HW
PPW = SC_PIX // NW
CHUNK = 160
NBUF = 3
N_CHUNKS = 12
GROUPS = CHUNK // 16

B_BLK = 8


def _tc_body(x_ref, idx_ref, sp_ref, ch_ref, out_ref):
    x = x_ref[...]
    idx = idx_ref[...]
    sp = sp_ref[...]
    ch = ch_ref[...]
    b_blk, hw = idx.shape
    lanes = jax.lax.broadcasted_iota(jnp.int32, (b_blk, hw, COLORS_PAD), 2)
    onehot = (idx[..., None] == lanes).astype(jnp.float32)
    base = x + sp[None, :, :]
    for b in range(b_blk):
        out_ref[b] = base[b] + jnp.dot(
            onehot[b], ch, preferred_element_type=jnp.float32)


def _tc_half(xf, idxf, sp128, ch128, n_imgs):
    grid = (n_imgs // B_BLK,)
    return pl.pallas_call(
        _tc_body,
        grid=grid,
        in_specs=[
            pl.BlockSpec((B_BLK, HW, D), lambda i: (i, 0, 0)),
            pl.BlockSpec((B_BLK, HW), lambda i: (i, 0)),
            pl.BlockSpec((HW, D), lambda i: (0, 0)),
            pl.BlockSpec((COLORS_PAD, D), lambda i: (0, 0)),
        ],
        out_specs=pl.BlockSpec((B_BLK, HW, D), lambda i: (i, 0, 0)),
        out_shape=jax.ShapeDtypeStruct((n_imgs, HW, D), jnp.float32),
    )(xf, idxf, sp128, ch128)


def _sc_kernel(x_hbm, idx_hbm, sp_hbm, ch_hbm, out_hbm,
               x_v0, x_v1, x_v2, i_v0, i_v1, i_v2, sp_v, ch_v,
               ld0, ld1, ld2, st0, st1, st2):
    x_bufs = (x_v0, x_v1, x_v2)
    i_bufs = (i_v0, i_v1, i_v2)
    ld_sems = (ld0, ld1, ld2)
    st_sems = (st0, st1, st2)

    wid = lax.axis_index("s") * 2 + lax.axis_index("c")
    base = wid * PPW
    last_start = base + PPW - CHUNK

    pltpu.sync_copy(sp_hbm, sp_v)
    pltpu.sync_copy(ch_hbm, ch_v)

    lane = lax.iota(jnp.int32, 16)

    def chunk_start(c):
        return lax.min(base + c * CHUNK, last_start)

    def start_load(c, b):
        pstart = chunk_start(c)
        pltpu.async_copy(x_hbm.at[pl.ds(pstart, CHUNK)],
                         x_bufs[b], ld_sems[b])
        pltpu.async_copy(idx_hbm.at[pl.ds(pstart, CHUNK)],
                         i_bufs[b], ld_sems[b])

    def wait_load(c, b):
        pstart = chunk_start(c)
        pltpu.make_async_copy(x_hbm.at[pl.ds(pstart, CHUNK)],
                              x_bufs[b], ld_sems[b]).wait()
        pltpu.make_async_copy(idx_hbm.at[pl.ds(pstart, CHUNK)],
                              i_bufs[b], ld_sems[b]).wait()

    def start_store(c, b):
        pstart = chunk_start(c)
        pltpu.async_copy(x_bufs[b],
                         out_hbm.at[pl.ds(pstart, CHUNK)], st_sems[b])

    def wait_store(c, b):
        pstart = chunk_start(c)
        pltpu.make_async_copy(x_bufs[b],
                              out_hbm.at[pl.ds(pstart, CHUNK)],
                              st_sems[b]).wait()

    for b in range(NBUF):
        start_load(b, b)

    def chunk_compute(pstart, x_v, idx_v):
        def group_body(g, _):
            p16 = g * 16
            cidx = idx_v[pl.ds(p16, 16)]
            for p in range(16):
                ploc = p16 + p
                prow = lax.rem(pstart + ploc, HW)
                sprow = prow // 2
                rcol = lax.rem(prow, 2) * HALF
                for j in range(HALF // 16):
                    o = j * 16
                    x_v[ploc, pl.ds(o, 16)] = (
                        x_v[ploc, pl.ds(o, 16)]
                        + sp_v[sprow, pl.ds(rcol + o, 16)])
                cbase = cidx[jnp.full((16,), p, jnp.int32)] * HALF
                for j in range(HALF // 16):
                    o = j * 16
                    cv = plsc.load_gather(ch_v, [cbase + o + lane])
                    x_v[ploc, pl.ds(HALF + o, 16)] = (
                        x_v[ploc, pl.ds(HALF + o, 16)] + cv)
            return 0

        lax.fori_loop(0, GROUPS, group_body, 0)

    def iter_body(k, _):
        for b in range(NBUF):
            c = k * NBUF + b
            wait_load(c, b)
            chunk_compute(chunk_start(c), x_bufs[b], i_bufs[b])
            start_store(c, b)

            @pl.when(c + NBUF < N_CHUNKS)
            def _():
                wait_store(c, b)
                start_load(c + NBUF, b)
        return 0

    lax.fori_loop(0, N_CHUNKS // NBUF, iter_body, 0)

    for b in range(NBUF):
        wait_store(N_CHUNKS - NBUF + b, b)


def _sc_half(xf, idxf, spf, chf):
    mesh = plsc.VectorSubcoreMesh(core_axis_name="c", subcore_axis_name="s")
    run = pl.kernel(
        _sc_kernel,
        jax.ShapeDtypeStruct((SC_PIX, D), jnp.float32),
        mesh=mesh,
        compiler_params=pltpu.CompilerParams(needs_layout_passes=False),
        scratch_types=[
            pltpu.VMEM((CHUNK, D), jnp.float32),
            pltpu.VMEM((CHUNK, D), jnp.float32),
            pltpu.VMEM((CHUNK, D), jnp.float32),
            pltpu.VMEM((CHUNK,), jnp.int32),
            pltpu.VMEM((CHUNK,), jnp.int32),
            pltpu.VMEM((CHUNK,), jnp.int32),
            pltpu.VMEM((HW // 2, D), jnp.float32),
            pltpu.VMEM((NUM_COLORS * HALF,), jnp.float32),
            pltpu.SemaphoreType.DMA,
            pltpu.SemaphoreType.DMA,
            pltpu.SemaphoreType.DMA,
            pltpu.SemaphoreType.DMA,
            pltpu.SemaphoreType.DMA,
            pltpu.SemaphoreType.DMA,
        ],
    )
    return run(xf, idxf, spf, chf)


def kernel(x, color_indices, spatial_pe, chromatic_pe):
    Bb, Hh, Ww, d = x.shape
    idx32 = color_indices.astype(jnp.int32)

    sp_half = spatial_pe[:Hh, :Ww, :].reshape(HW, HALF)

    sp128 = jnp.concatenate(
        [sp_half, jnp.zeros((HW, HALF), jnp.float32)], axis=-1)
    ch128 = jnp.zeros((COLORS_PAD, D), jnp.float32)
    ch128 = ch128.at[:NUM_COLORS, HALF:].set(chromatic_pe)
    out_tc = _tc_half(
        x[SC_IMGS:].reshape(TC_IMGS, HW, D),
        idx32[SC_IMGS:].reshape(TC_IMGS, HW),
        sp128, ch128, TC_IMGS)

    out_sc = _sc_half(
        x[:SC_IMGS].reshape(SC_PIX, D),
        idx32[:SC_IMGS].reshape(SC_PIX),
        sp_half.reshape(HW // 2, D),
        chromatic_pe.reshape(NUM_COLORS * HALF))

    out = jnp.concatenate(
        [out_sc.reshape(SC_IMGS, HW, D),
         out_tc.reshape(TC_IMGS, HW, D)], axis=0)
    return out.reshape(Bb, Hh, Ww, d)


# speedup vs baseline: 1.3588x; 1.0011x over previous
import jax
import jax.numpy as jnp
from jax import lax
from jax.experimental import pallas as pl
from jax.experimental.pallas import tpu as pltpu
from jax.experimental.pallas import tpu_sc as plsc

D = 128
HALF = 64
HW = 900
NUM_COLORS = 10
COLORS_PAD = 16
NW = 32

TC_IMGS = 192
SC_IMGS = 64
SC_PIX = SC_IMGS * HW
PPW = SC_PIX // NW
CHUNK = 160
NBUF = 3
N_CHUNKS = 12
GROUPS = CHUNK // 16

B_BLK = 8


def _tc_body(x_ref, idx_ref, sp_ref, ch_ref, out_ref):
    x = x_ref[...]
    idx = idx_ref[...]
    sp = sp_ref[...]
    ch = ch_ref[...]
    b_blk, hw = idx.shape
    lanes = jax.lax.broadcasted_iota(jnp.int32, (b_blk, hw, COLORS_PAD), 2)
    onehot = (idx[..., None] == lanes).astype(jnp.float32)
    base = x + sp[None, :, :]
    for b in range(b_blk):
        out_ref[b] = base[b] + jnp.dot(
            onehot[b], ch, preferred_element_type=jnp.float32)


def _tc_half(xf, idxf, sp128, ch128, n_imgs):
    grid = (n_imgs // B_BLK,)
    return pl.pallas_call(
        _tc_body,
        grid=grid,
        in_specs=[
            pl.BlockSpec((B_BLK, HW, D), lambda i: (i, 0, 0)),
            pl.BlockSpec((B_BLK, HW), lambda i: (i, 0)),
            pl.BlockSpec((HW, D), lambda i: (0, 0)),
            pl.BlockSpec((COLORS_PAD, D), lambda i: (0, 0)),
        ],
        out_specs=pl.BlockSpec((B_BLK, HW, D), lambda i: (i, 0, 0)),
        out_shape=jax.ShapeDtypeStruct((n_imgs, HW, D), jnp.float32),
    )(xf, idxf, sp128, ch128)


def _sc_kernel(x_hbm, idx_hbm, sp_hbm, ch_hbm, out_hbm,
               x_v0, x_v1, x_v2, i_v0, i_v1, i_v2, sp_v, ch_v,
               ld0, ld1, ld2, st0, st1, st2):
    x_bufs = (x_v0, x_v1, x_v2)
    i_bufs = (i_v0, i_v1, i_v2)
    ld_sems = (ld0, ld1, ld2)
    st_sems = (st0, st1, st2)

    wid = lax.axis_index("s") * 2 + lax.axis_index("c")
    base = wid * PPW
    last_start = base + PPW - CHUNK

    pltpu.sync_copy(sp_hbm, sp_v)
    pltpu.sync_copy(ch_hbm, ch_v)

    lane = lax.iota(jnp.int32, 16)

    def chunk_start(c):
        return lax.min(base + c * CHUNK, last_start)

    def start_load(c, b):
        pstart = chunk_start(c)
        pltpu.async_copy(x_hbm.at[pl.ds(pstart, CHUNK)],
                         x_bufs[b], ld_sems[b])
        pltpu.async_copy(idx_hbm.at[pl.ds(pstart, CHUNK)],
                         i_bufs[b], ld_sems[b])

    def wait_load(c, b):
        pstart = chunk_start(c)
        pltpu.make_async_copy(x_hbm.at[pl.ds(pstart, CHUNK)],
                              x_bufs[b], ld_sems[b]).wait()
        pltpu.make_async_copy(idx_hbm.at[pl.ds(pstart, CHUNK)],
                              i_bufs[b], ld_sems[b]).wait()

    def start_store(c, b):
        pstart = chunk_start(c)
        pltpu.async_copy(x_bufs[b],
                         out_hbm.at[pl.ds(pstart, CHUNK)], st_sems[b])

    def wait_store(c, b):
        pstart = chunk_start(c)
        pltpu.make_async_copy(x_bufs[b],
                              out_hbm.at[pl.ds(pstart, CHUNK)],
                              st_sems[b]).wait()

    for b in range(NBUF):
        start_load(b, b)

    def chunk_compute(pstart, x_v, idx_v):
        def group_body(g, _):
            p16 = g * 16
            cidx = idx_v[pl.ds(p16, 16)]
            for p in range(16):
                ploc = p16 + p
                prow = lax.rem(pstart + ploc, HW)
                sprow = prow // 2
                rcol = lax.rem(prow, 2) * HALF
                for j in range(HALF // 16):
                    o = j * 16
                    x_v[ploc, pl.ds(o, 16)] = (
                        x_v[ploc, pl.ds(o, 16)]
                        + sp_v[sprow, pl.ds(rcol + o, 16)])
                cbase = cidx[jnp.full((16,), p, jnp.int32)] * HALF
                for j in range(HALF // 16):
                    o = j * 16
                    cv = plsc.load_gather(ch_v, [cbase + o + lane])
                    x_v[ploc, pl.ds(HALF + o, 16)] = (
                        x_v[ploc, pl.ds(HALF + o, 16)] + cv)
            return 0

        lax.fori_loop(0, GROUPS, group_body, 0)

    def iter_body(k, _):
        for b in range(NBUF):
            c = k * NBUF + b
            wait_load(c, b)
            chunk_compute(chunk_start(c), x_bufs[b], i_bufs[b])
            start_store(c, b)

            @pl.when(c + NBUF < N_CHUNKS)
            def _():
                wait_store(c, b)
                start_load(c + NBUF, b)
        return 0

    lax.fori_loop(0, N_CHUNKS // NBUF, iter_body, 0)

    for b in range(NBUF):
        wait_store(N_CHUNKS - NBUF + b, b)


def _sc_half(xf, idxf, spf, chf):
    mesh = plsc.VectorSubcoreMesh(core_axis_name="c", subcore_axis_name="s")
    run = pl.kernel(
        _sc_kernel,
        jax.ShapeDtypeStruct((SC_PIX, D), jnp.float32),
        mesh=mesh,
        compiler_params=pltpu.CompilerParams(needs_layout_passes=False),
        scratch_types=[
            pltpu.VMEM((CHUNK, D), jnp.float32),
            pltpu.VMEM((CHUNK, D), jnp.float32),
            pltpu.VMEM((CHUNK, D), jnp.float32),
            pltpu.VMEM((CHUNK,), jnp.int32),
            pltpu.VMEM((CHUNK,), jnp.int32),
            pltpu.VMEM((CHUNK,), jnp.int32),
            pltpu.VMEM((HW // 2, D), jnp.float32),
            pltpu.VMEM((NUM_COLORS * HALF,), jnp.float32),
            pltpu.SemaphoreType.DMA,
            pltpu.SemaphoreType.DMA,
            pltpu.SemaphoreType.DMA,
            pltpu.SemaphoreType.DMA,
            pltpu.SemaphoreType.DMA,
            pltpu.SemaphoreType.DMA,
        ],
    )
    return run(xf, idxf, spf, chf)


def kernel(x, color_indices, spatial_pe, chromatic_pe):
    Bb, Hh, Ww, d = x.shape
    idx32 = color_indices.astype(jnp.int32)

    sp_half = spatial_pe[:Hh, :Ww, :].reshape(HW, HALF)

    out_sc = _sc_half(
        x[:SC_IMGS].reshape(SC_PIX, D),
        idx32[:SC_IMGS].reshape(SC_PIX),
        sp_half.reshape(HW // 2, D),
        chromatic_pe.reshape(NUM_COLORS * HALF))

    sp128 = jnp.concatenate(
        [sp_half, jnp.zeros((HW, HALF), jnp.float32)], axis=-1)
    ch128 = jnp.zeros((COLORS_PAD, D), jnp.float32)
    ch128 = ch128.at[:NUM_COLORS, HALF:].set(chromatic_pe)
    out_tc = _tc_half(
        x[SC_IMGS:].reshape(TC_IMGS, HW, D),
        idx32[SC_IMGS:].reshape(TC_IMGS, HW),
        sp128, ch128, TC_IMGS)

    out = jnp.concatenate(
        [out_sc.reshape(SC_IMGS, HW, D),
         out_tc.reshape(TC_IMGS, HW, D)], axis=0)
    return out.reshape(Bb, Hh, Ww, d)
